# Initial kernel scaffold; baseline (speedup 1.0000x reference)
import jax
import jax.numpy as jnp
from jax.experimental import pallas as pl


def kernel(query, input_flatten, W_so, b_so, W_aw, b_aw, W_tso, b_tso, W_taw, b_taw, W_vp, b_vp):
    raise NotImplementedError("write your pallas kernel here")


# speedup vs baseline: 1.1193x; 1.1193x over previous
import functools

import jax
import jax.numpy as jnp
from jax.experimental import pallas as pl
from jax.experimental.pallas import tpu as pltpu

_T = 36
_LQ = 3060
_C = 256
_H = 8
_L = 4
_NCP = 4
_NTP = 2
_TW = 2

_BLK = 1080


def _body(q_ref, x_ref, wq_ref, bq_ref, wv_ref, bv_ref, g_ref,
          val_ref, cso_ref, tso_ref, awc_ref, awt_ref):
    x = x_ref[:]
    val_ref[:] = (
        jnp.dot(x, wv_ref[:], preferred_element_type=jnp.float32) + bv_ref[:]
    )
    q = q_ref[:]
    p = jnp.dot(q, wq_ref[:], preferred_element_type=jnp.float32) + bq_ref[:]
    cso_ref[:] = p[:, :256]
    tso_ref[:] = p[:, 256:512]
    caw = p[:, 512:640]
    taw = p[:, 640:768]
    m = jnp.max(jnp.maximum(caw, taw), axis=1, keepdims=True)
    ea = jnp.exp(caw - m)
    eb = jnp.exp(taw - m)
    s = jnp.dot(ea + eb, g_ref[:], preferred_element_type=jnp.float32)
    r = 1.0 / s
    awc_ref[:] = ea * r
    awt_ref[:] = eb * r


@functools.partial(jax.jit)
def kernel(query, input_flatten, W_so, b_so, W_aw, b_aw, W_tso, b_tso,
           W_taw, b_taw, W_vp, b_vp):
    Tn, Lq, Cd = query.shape
    _, Lin, _ = input_flatten.shape
    n_q = Tn * Lq
    n_in = Tn * Lin

    q_flat = query.reshape(n_q, Cd)
    x_flat = input_flatten.reshape(n_in, Cd)

    wq = jnp.concatenate([W_so, W_tso, W_aw, W_taw], axis=1)
    bq = jnp.concatenate([b_so, b_tso, b_aw, b_taw])[None, :]
    bv = b_vp[None, :]
    lane = jnp.arange(128) // 16
    g = (lane[:, None] == lane[None, :]).astype(jnp.float32)

    grid = (n_q // _BLK,)
    row_spec = lambda w: pl.BlockSpec((_BLK, w), lambda i: (i, 0))
    full_spec = lambda a, b: pl.BlockSpec((a, b), lambda i: (0, 0))

    outs = pl.pallas_call(
        _body,
        grid=grid,
        in_specs=[
            row_spec(Cd),
            row_spec(Cd),
            full_spec(Cd, 768),
            full_spec(1, 768),
            full_spec(Cd, Cd),
            full_spec(1, Cd),
            full_spec(128, 128),
        ],
        out_specs=[
            row_spec(Cd),
            row_spec(Cd),
            row_spec(Cd),
            row_spec(128),
            row_spec(128),
        ],
        out_shape=[
            jax.ShapeDtypeStruct((n_in, Cd), jnp.float32),
            jax.ShapeDtypeStruct((n_q, Cd), jnp.float32),
            jax.ShapeDtypeStruct((n_q, Cd), jnp.float32),
            jax.ShapeDtypeStruct((n_q, 128), jnp.float32),
            jax.ShapeDtypeStruct((n_q, 128), jnp.float32),
        ],
        compiler_params=pltpu.CompilerParams(
            dimension_semantics=("arbitrary",),
        ),
    )(q_flat, x_flat, wq, bq, W_vp, bv, g)

    val_f, cso_f, tso_f, awc_f, awt_f = outs
    value = val_f.reshape(Tn, Lin, _H, Cd // _H)
    cso = cso_f.reshape(Tn, Lq, _H, _L, _NCP, 2)
    tso = tso_f.reshape(Tn, Lq, _H, _TW * _L, _NTP, 2)
    aw_curr = awc_f.reshape(Tn, Lq, _H, _L, _NCP)
    aw_temp = awt_f.reshape(Tn, Lq, _H, _TW * _L, _NTP)
    return (value, cso, tso, aw_curr, aw_temp)


# speedup vs baseline: 1.4727x; 1.3157x over previous
import functools

import jax
import jax.numpy as jnp
from jax.experimental import pallas as pl
from jax.experimental.pallas import tpu as pltpu

_T = 36
_LQ = 3060
_C = 256
_H = 8
_L = 4
_NCP = 4
_NTP = 2
_TW = 2

_BQ = 512

_DN = (((0,), (1,)), ((), ()))


def _body(q_ref, x_ref, wq_ref, bq_ref, wv_ref, bv_ref, g_ref,
          val_ref, cso_ref, tso_ref, awc_ref, awt_ref):
    x = x_ref[0]
    val_ref[0] = (
        jax.lax.dot_general(wv_ref[:], x, _DN,
                            preferred_element_type=jnp.float32)
        + bv_ref[:]
    )
    q = q_ref[0]
    p = jax.lax.dot_general(wq_ref[:], q, _DN,
                            preferred_element_type=jnp.float32) + bq_ref[:]
    cso_ref[0] = p[:256, :]
    tso_ref[0] = p[256:512, :]
    caw = p[512:640, :]
    taw = p[640:768, :]
    m = jnp.max(jnp.maximum(caw, taw), axis=0, keepdims=True)
    ea = jnp.exp(caw - m)
    eb = jnp.exp(taw - m)
    s = jnp.dot(g_ref[:], ea + eb, preferred_element_type=jnp.float32)
    r = 1.0 / s
    awc_ref[0] = ea * r
    awt_ref[0] = eb * r


@functools.partial(jax.jit)
def kernel(query, input_flatten, W_so, b_so, W_aw, b_aw, W_tso, b_tso,
           W_taw, b_taw, W_vp, b_vp):
    Tn, Lq, Cd = query.shape
    _, Lin, _ = input_flatten.shape

    wq = jnp.concatenate([W_so, W_tso, W_aw, W_taw], axis=1)
    bq = jnp.concatenate([b_so, b_tso, b_aw, b_taw])[:, None]
    bv = b_vp[:, None]
    lane = jnp.arange(128) // 16
    g = (lane[:, None] == lane[None, :]).astype(jnp.float32)

    nbq = pl.cdiv(Lq, _BQ)
    grid = (Tn, nbq)
    act_spec = pl.BlockSpec((1, _BQ, Cd), lambda t, j: (t, j, 0))
    out_spec = lambda f: pl.BlockSpec((1, f, _BQ), lambda t, j: (t, 0, j))
    full_spec = lambda a, b: pl.BlockSpec((a, b), lambda t, j: (0, 0))

    outs = pl.pallas_call(
        _body,
        grid=grid,
        in_specs=[
            act_spec,
            act_spec,
            full_spec(Cd, 768),
            full_spec(768, 1),
            full_spec(Cd, Cd),
            full_spec(Cd, 1),
            full_spec(128, 128),
        ],
        out_specs=[
            out_spec(Cd),
            out_spec(Cd),
            out_spec(Cd),
            out_spec(128),
            out_spec(128),
        ],
        out_shape=[
            jax.ShapeDtypeStruct((Tn, Cd, Lin), jnp.float32),
            jax.ShapeDtypeStruct((Tn, Cd, Lq), jnp.float32),
            jax.ShapeDtypeStruct((Tn, Cd, Lq), jnp.float32),
            jax.ShapeDtypeStruct((Tn, 128, Lq), jnp.float32),
            jax.ShapeDtypeStruct((Tn, 128, Lq), jnp.float32),
        ],
        compiler_params=pltpu.CompilerParams(
            dimension_semantics=("arbitrary", "arbitrary"),
        ),
    )(query, input_flatten, wq, bq, W_vp, bv, g)

    val_t, cso_t, tso_t, awc_t, awt_t = outs
    value = val_t.reshape(Tn, _H, Cd // _H, Lin).transpose(0, 3, 1, 2)
    cso = cso_t.reshape(Tn, _H, _L, _NCP, 2, Lq).transpose(0, 5, 1, 2, 3, 4)
    tso = tso_t.reshape(Tn, _H, _TW * _L, _NTP, 2, Lq).transpose(0, 5, 1, 2, 3, 4)
    aw_curr = awc_t.reshape(Tn, _H, _L, _NCP, Lq).transpose(0, 4, 1, 2, 3)
    aw_temp = awt_t.reshape(Tn, _H, _TW * _L, _NTP, Lq).transpose(0, 4, 1, 2, 3)
    return (value, cso, tso, aw_curr, aw_temp)


# speedup vs baseline: 1.6128x; 1.0952x over previous
import functools

import jax
import jax.numpy as jnp
from jax.experimental import pallas as pl
from jax.experimental.pallas import tpu as pltpu

_T = 36
_LQ = 3060
_C = 256
_H = 8
_L = 4
_NCP = 4
_NTP = 2
_TW = 2

_BQ = 1536

_DN = (((0,), (1,)), ((), ()))


def _body(q_ref, x_ref, wq_ref, bq_ref, wv_ref, bv_ref, g_ref,
          val_ref, cso_ref, tso_ref, awc_ref, awt_ref):
    x = x_ref[0]
    val_ref[0] = (
        jax.lax.dot_general(wv_ref[:], x, _DN,
                            preferred_element_type=jnp.float32)
        + bv_ref[:]
    )
    q = q_ref[0]
    p = jax.lax.dot_general(wq_ref[:], q, _DN,
                            preferred_element_type=jnp.float32) + bq_ref[:]
    cso_ref[0] = p[:256, :]
    tso_ref[0] = p[256:512, :]
    caw = p[512:640, :]
    taw = p[640:768, :]
    m = jnp.max(jnp.maximum(caw, taw), axis=0, keepdims=True)
    ea = jnp.exp(caw - m)
    eb = jnp.exp(taw - m)
    s = jnp.dot(g_ref[:], ea + eb, preferred_element_type=jnp.float32)
    r = 1.0 / s
    awc_ref[0] = ea * r
    awt_ref[0] = eb * r


@functools.partial(jax.jit)
def kernel(query, input_flatten, W_so, b_so, W_aw, b_aw, W_tso, b_tso,
           W_taw, b_taw, W_vp, b_vp):
    Tn, Lq, Cd = query.shape
    _, Lin, _ = input_flatten.shape

    wq = jnp.concatenate([W_so, W_tso, W_aw, W_taw], axis=1)
    bq = jnp.concatenate([b_so, b_tso, b_aw, b_taw])[:, None]
    bv = b_vp[:, None]
    lane = jnp.arange(128) // 16
    g = (lane[:, None] == lane[None, :]).astype(jnp.float32)

    nbq = pl.cdiv(Lq, _BQ)
    grid = (Tn, nbq)
    act_spec = pl.BlockSpec((1, _BQ, Cd), lambda t, j: (t, j, 0))
    out_spec = lambda f: pl.BlockSpec((1, f, _BQ), lambda t, j: (t, 0, j))
    full_spec = lambda a, b: pl.BlockSpec((a, b), lambda t, j: (0, 0))

    outs = pl.pallas_call(
        _body,
        grid=grid,
        in_specs=[
            act_spec,
            act_spec,
            full_spec(Cd, 768),
            full_spec(768, 1),
            full_spec(Cd, Cd),
            full_spec(Cd, 1),
            full_spec(128, 128),
        ],
        out_specs=[
            out_spec(Cd),
            out_spec(Cd),
            out_spec(Cd),
            out_spec(128),
            out_spec(128),
        ],
        out_shape=[
            jax.ShapeDtypeStruct((Tn, Cd, Lin), jnp.float32),
            jax.ShapeDtypeStruct((Tn, Cd, Lq), jnp.float32),
            jax.ShapeDtypeStruct((Tn, Cd, Lq), jnp.float32),
            jax.ShapeDtypeStruct((Tn, 128, Lq), jnp.float32),
            jax.ShapeDtypeStruct((Tn, 128, Lq), jnp.float32),
        ],
        compiler_params=pltpu.CompilerParams(
            dimension_semantics=("arbitrary", "arbitrary"),
        ),
    )(query, input_flatten, wq, bq, W_vp, bv, g)

    val_t, cso_t, tso_t, awc_t, awt_t = outs
    value = val_t.reshape(Tn, _H, Cd // _H, Lin).transpose(0, 3, 1, 2)
    cso = cso_t.reshape(Tn, _H, _L, _NCP, 2, Lq).transpose(0, 5, 1, 2, 3, 4)
    tso = tso_t.reshape(Tn, _H, _TW * _L, _NTP, 2, Lq).transpose(0, 5, 1, 2, 3, 4)
    aw_curr = awc_t.reshape(Tn, _H, _L, _NCP, Lq).transpose(0, 4, 1, 2, 3)
    aw_temp = awt_t.reshape(Tn, _H, _TW * _L, _NTP, Lq).transpose(0, 4, 1, 2, 3)
    return (value, cso, tso, aw_curr, aw_temp)


# speedup vs baseline: 1.6203x; 1.0047x over previous
import functools

import jax
import jax.numpy as jnp
from jax.experimental import pallas as pl
from jax.experimental.pallas import tpu as pltpu

_T = 36
_LQ = 3060
_C = 256
_H = 8
_L = 4
_NCP = 4
_NTP = 2
_TW = 2

_BQ = 3072

_DN = (((0,), (1,)), ((), ()))


def _body(q_ref, x_ref, wq_ref, bq_ref, wv_ref, bv_ref, g_ref,
          val_ref, cso_ref, tso_ref, awc_ref, awt_ref):
    x = x_ref[0]
    val_ref[0] = (
        jax.lax.dot_general(wv_ref[:], x, _DN,
                            preferred_element_type=jnp.float32)
        + bv_ref[:]
    )
    q = q_ref[0]
    p = jax.lax.dot_general(wq_ref[:], q, _DN,
                            preferred_element_type=jnp.float32) + bq_ref[:]
    cso_ref[0] = p[:256, :]
    tso_ref[0] = p[256:512, :]
    caw = p[512:640, :]
    taw = p[640:768, :]
    m = jnp.max(jnp.maximum(caw, taw), axis=0, keepdims=True)
    ea = jnp.exp(caw - m)
    eb = jnp.exp(taw - m)
    s = jnp.dot(g_ref[:], ea + eb, preferred_element_type=jnp.float32)
    r = 1.0 / s
    awc_ref[0] = ea * r
    awt_ref[0] = eb * r


@functools.partial(jax.jit)
def kernel(query, input_flatten, W_so, b_so, W_aw, b_aw, W_tso, b_tso,
           W_taw, b_taw, W_vp, b_vp):
    Tn, Lq, Cd = query.shape
    _, Lin, _ = input_flatten.shape

    wq = jnp.concatenate([W_so, W_tso, W_aw, W_taw], axis=1)
    bq = jnp.concatenate([b_so, b_tso, b_aw, b_taw])[:, None]
    bv = b_vp[:, None]
    lane = jnp.arange(128) // 16
    g = (lane[:, None] == lane[None, :]).astype(jnp.float32)

    nbq = pl.cdiv(Lq, _BQ)
    grid = (Tn, nbq)
    act_spec = pl.BlockSpec((1, _BQ, Cd), lambda t, j: (t, j, 0))
    out_spec = lambda f: pl.BlockSpec((1, f, _BQ), lambda t, j: (t, 0, j))
    full_spec = lambda a, b: pl.BlockSpec((a, b), lambda t, j: (0, 0))

    outs = pl.pallas_call(
        _body,
        grid=grid,
        in_specs=[
            act_spec,
            act_spec,
            full_spec(Cd, 768),
            full_spec(768, 1),
            full_spec(Cd, Cd),
            full_spec(Cd, 1),
            full_spec(128, 128),
        ],
        out_specs=[
            out_spec(Cd),
            out_spec(Cd),
            out_spec(Cd),
            out_spec(128),
            out_spec(128),
        ],
        out_shape=[
            jax.ShapeDtypeStruct((Tn, Cd, Lin), jnp.float32),
            jax.ShapeDtypeStruct((Tn, Cd, Lq), jnp.float32),
            jax.ShapeDtypeStruct((Tn, Cd, Lq), jnp.float32),
            jax.ShapeDtypeStruct((Tn, 128, Lq), jnp.float32),
            jax.ShapeDtypeStruct((Tn, 128, Lq), jnp.float32),
        ],
        compiler_params=pltpu.CompilerParams(
            dimension_semantics=("arbitrary", "arbitrary"),
        ),
    )(query, input_flatten, wq, bq, W_vp, bv, g)

    val_t, cso_t, tso_t, awc_t, awt_t = outs
    value = val_t.reshape(Tn, _H, Cd // _H, Lin).transpose(0, 3, 1, 2)
    cso = cso_t.reshape(Tn, _H, _L, _NCP, 2, Lq).transpose(0, 5, 1, 2, 3, 4)
    tso = tso_t.reshape(Tn, _H, _TW * _L, _NTP, 2, Lq).transpose(0, 5, 1, 2, 3, 4)
    aw_curr = awc_t.reshape(Tn, _H, _L, _NCP, Lq).transpose(0, 4, 1, 2, 3)
    aw_temp = awt_t.reshape(Tn, _H, _TW * _L, _NTP, Lq).transpose(0, 4, 1, 2, 3)
    return (value, cso, tso, aw_curr, aw_temp)


# speedup vs baseline: 1.6214x; 1.0006x over previous
import functools

import jax
import jax.numpy as jnp
from jax.experimental import pallas as pl
from jax.experimental.pallas import tpu as pltpu

_T = 36
_LQ = 3060
_C = 256
_H = 8
_L = 4
_NCP = 4
_NTP = 2
_TW = 2

_BQ = 3072

_DN = (((0,), (1,)), ((), ()))


def _body(q_ref, x_ref, wq_ref, bq_ref, wv_ref, bv_ref, g_ref,
          val_ref, cso_ref, tso_ref, awc_ref, awt_ref):
    x = x_ref[0]
    val_ref[0] = (
        jax.lax.dot_general(wv_ref[:], x, _DN,
                            preferred_element_type=jnp.float32)
        + bv_ref[:]
    )
    q = q_ref[0]
    p = jax.lax.dot_general(wq_ref[:], q, _DN,
                            preferred_element_type=jnp.float32) + bq_ref[:]
    cso_ref[0] = p[:256, :]
    tso_ref[0] = p[256:512, :]
    caw = p[512:640, :]
    taw = p[640:768, :]
    m = jnp.max(jnp.maximum(caw, taw), axis=0, keepdims=True)
    ea = jnp.exp(caw - m)
    eb = jnp.exp(taw - m)
    s = jnp.dot(g_ref[:], ea + eb, preferred_element_type=jnp.float32)
    r = 1.0 / s
    awc_ref[0] = ea * r
    awt_ref[0] = eb * r


@functools.partial(jax.jit)
def kernel(query, input_flatten, W_so, b_so, W_aw, b_aw, W_tso, b_tso,
           W_taw, b_taw, W_vp, b_vp):
    Tn, Lq, Cd = query.shape
    _, Lin, _ = input_flatten.shape

    wq = jnp.concatenate([W_so, W_tso, W_aw, W_taw], axis=1)
    bq = jnp.concatenate([b_so, b_tso, b_aw, b_taw])[:, None]
    bv = b_vp[:, None]
    lane = jnp.arange(128) // 16
    g = (lane[:, None] == lane[None, :]).astype(jnp.float32)

    nbq = pl.cdiv(Lq, _BQ)
    grid = (Tn, nbq)
    act_spec = pl.BlockSpec((1, _BQ, Cd), lambda t, j: (t, j, 0))
    out_spec = lambda f: pl.BlockSpec((1, f, _BQ), lambda t, j: (t, 0, j))
    full_spec = lambda a, b: pl.BlockSpec((a, b), lambda t, j: (0, 0))

    outs = pl.pallas_call(
        _body,
        grid=grid,
        in_specs=[
            act_spec,
            act_spec,
            full_spec(Cd, 768),
            full_spec(768, 1),
            full_spec(Cd, Cd),
            full_spec(Cd, 1),
            full_spec(128, 128),
        ],
        out_specs=[
            out_spec(Cd),
            out_spec(Cd),
            out_spec(Cd),
            out_spec(128),
            out_spec(128),
        ],
        out_shape=[
            jax.ShapeDtypeStruct((Tn, Cd, Lin), jnp.float32),
            jax.ShapeDtypeStruct((Tn, Cd, Lq), jnp.float32),
            jax.ShapeDtypeStruct((Tn, Cd, Lq), jnp.float32),
            jax.ShapeDtypeStruct((Tn, 128, Lq), jnp.float32),
            jax.ShapeDtypeStruct((Tn, 128, Lq), jnp.float32),
        ],
        compiler_params=pltpu.CompilerParams(
            dimension_semantics=("parallel", "parallel"),
        ),
    )(query, input_flatten, wq, bq, W_vp, bv, g)

    val_t, cso_t, tso_t, awc_t, awt_t = outs
    value = val_t.reshape(Tn, _H, Cd // _H, Lin).transpose(0, 3, 1, 2)
    cso = cso_t.reshape(Tn, _H, _L, _NCP, 2, Lq).transpose(0, 5, 1, 2, 3, 4)
    tso = tso_t.reshape(Tn, _H, _TW * _L, _NTP, 2, Lq).transpose(0, 5, 1, 2, 3, 4)
    aw_curr = awc_t.reshape(Tn, _H, _L, _NCP, Lq).transpose(0, 4, 1, 2, 3)
    aw_temp = awt_t.reshape(Tn, _H, _TW * _L, _NTP, Lq).transpose(0, 4, 1, 2, 3)
    return (value, cso, tso, aw_curr, aw_temp)


# speedup vs baseline: 1.6229x; 1.0010x over previous
import functools

import jax
import jax.numpy as jnp
from jax.experimental import pallas as pl
from jax.experimental.pallas import tpu as pltpu

_T = 36
_LQ = 3060
_C = 256
_H = 8
_L = 4
_NCP = 4
_NTP = 2
_TW = 2

_BQ = 3072
_SPLIT = 2944

_DN = (((0,), (1,)), ((), ()))

_ROWS = ((0, 256), (256, 512), (512, 768), (768, 896), (896, 1024))


def _out_copies(hbm_refs, buf, tail, sem, par, t):
    cps = []
    for (r0, r1), hbm in zip(_ROWS, hbm_refs):
        cps.append(pltpu.make_async_copy(
            buf.at[par, pl.ds(r0, r1 - r0), pl.ds(0, _SPLIT)],
            hbm.at[t, :, pl.ds(0, _SPLIT)],
            sem.at[par],
        ))
        cps.append(pltpu.make_async_copy(
            tail.at[par, pl.ds(r0, r1 - r0), :],
            hbm.at[t, :, pl.ds(_SPLIT, _LQ - _SPLIT)],
            sem.at[par],
        ))
    return cps


def _body(q_ref, x_ref, wq_ref, bq_ref, wv_ref, bv_ref, g_ref,
          val_hbm, cso_hbm, tso_hbm, awc_hbm, awt_hbm, buf, tail, sem):
    t = pl.program_id(0)
    nt = pl.num_programs(0)
    par = jax.lax.rem(t, 2)
    outs = (val_hbm, cso_hbm, tso_hbm, awc_hbm, awt_hbm)

    @pl.when(t >= 2)
    def _wait_prev():
        for c in _out_copies(outs, buf, tail, sem, par, t - 2):
            c.wait()

    def put(r0, n, arr):
        buf[par, pl.ds(r0, n), :] = arr
        tail[par, pl.ds(r0, n), :] = arr[:, _SPLIT:_LQ]

    x = x_ref[0]
    put(0, 256,
        jax.lax.dot_general(wv_ref[:], x, _DN,
                            preferred_element_type=jnp.float32) + bv_ref[:])
    q = q_ref[0]
    p = jax.lax.dot_general(wq_ref[:], q, _DN,
                            preferred_element_type=jnp.float32) + bq_ref[:]
    put(256, 512, p[:512, :])
    caw = p[512:640, :]
    taw = p[640:768, :]
    m = jnp.max(jnp.maximum(caw, taw), axis=0, keepdims=True)
    ea = jnp.exp(caw - m)
    eb = jnp.exp(taw - m)
    s = jnp.dot(g_ref[:], ea + eb, preferred_element_type=jnp.float32)
    r = 1.0 / s
    put(768, 128, ea * r)
    put(896, 128, eb * r)

    for c in _out_copies(outs, buf, tail, sem, par, t):
        c.start()

    @pl.when(t == nt - 1)
    def _drain_self():
        for c in _out_copies(outs, buf, tail, sem, par, t):
            c.wait()

    @pl.when((t == nt - 1) & (t >= 1))
    def _drain_other():
        for c in _out_copies(outs, buf, tail, sem, 1 - par, t - 1):
            c.wait()


@functools.partial(jax.jit)
def kernel(query, input_flatten, W_so, b_so, W_aw, b_aw, W_tso, b_tso,
           W_taw, b_taw, W_vp, b_vp):
    Tn, Lq, Cd = query.shape
    _, Lin, _ = input_flatten.shape

    wq = jnp.concatenate([W_so, W_tso, W_aw, W_taw], axis=1)
    bq = jnp.concatenate([b_so, b_tso, b_aw, b_taw])[:, None]
    bv = b_vp[:, None]
    lane = jnp.arange(128) // 16
    g = (lane[:, None] == lane[None, :]).astype(jnp.float32)

    grid = (Tn,)
    act_spec = pl.BlockSpec((1, _BQ, Cd), lambda t: (t, 0, 0))
    any_spec = pl.BlockSpec(memory_space=pl.ANY)
    full_spec = lambda a, b: pl.BlockSpec((a, b), lambda t: (0, 0))

    outs = pl.pallas_call(
        _body,
        grid=grid,
        in_specs=[
            act_spec,
            act_spec,
            full_spec(Cd, 768),
            full_spec(768, 1),
            full_spec(Cd, Cd),
            full_spec(Cd, 1),
            full_spec(128, 128),
        ],
        out_specs=[any_spec] * 5,
        out_shape=[
            jax.ShapeDtypeStruct((Tn, Cd, Lin), jnp.float32),
            jax.ShapeDtypeStruct((Tn, Cd, Lq), jnp.float32),
            jax.ShapeDtypeStruct((Tn, Cd, Lq), jnp.float32),
            jax.ShapeDtypeStruct((Tn, 128, Lq), jnp.float32),
            jax.ShapeDtypeStruct((Tn, 128, Lq), jnp.float32),
        ],
        scratch_shapes=[
            pltpu.VMEM((2, 1024, _BQ), jnp.float32),
            pltpu.VMEM((2, 1024, _LQ - _SPLIT), jnp.float32),
            pltpu.SemaphoreType.DMA((2,)),
        ],
        compiler_params=pltpu.CompilerParams(
            dimension_semantics=("arbitrary",),
        ),
    )(query, input_flatten, wq, bq, W_vp, bv, g)

    val_t, cso_t, tso_t, awc_t, awt_t = outs
    value = val_t.reshape(Tn, _H, Cd // _H, Lin).transpose(0, 3, 1, 2)
    cso = cso_t.reshape(Tn, _H, _L, _NCP, 2, Lq).transpose(0, 5, 1, 2, 3, 4)
    tso = tso_t.reshape(Tn, _H, _TW * _L, _NTP, 2, Lq).transpose(0, 5, 1, 2, 3, 4)
    aw_curr = awc_t.reshape(Tn, _H, _L, _NCP, Lq).transpose(0, 4, 1, 2, 3)
    aw_temp = awt_t.reshape(Tn, _H, _TW * _L, _NTP, Lq).transpose(0, 4, 1, 2, 3)
    return (value, cso, tso, aw_curr, aw_temp)
